# Initial kernel scaffold; baseline (speedup 1.0000x reference)
import jax
import jax.numpy as jnp
from jax.experimental import pallas as pl


def kernel(sparse_feat, dense_feat, linear_emb, emb, lin_W, lin_b):
    raise NotImplementedError("write your pallas kernel here")


# speedup vs baseline: 176.0183x; 176.0183x over previous
import functools

import jax
import jax.numpy as jnp
from jax import lax
from jax.experimental import pallas as pl
from jax.experimental.pallas import tpu as pltpu
from jax.experimental.pallas import tpu_sc as plsc

FIELDS = 26
FS = 100
EMB_D = 32
STRIDE = FS + 1
WIDTH = FIELDS * STRIDE
CW = FIELDS * FS
WPAD = 2688
HP = 2816
TRASH = CW
LANES = 16
N_CHUNKS = WPAD // LANES
N_HCHUNKS = HP // LANES
NW = 32


def _tc_targets(sp_t):
    W, B = sp_t.shape
    BLK = 128
    n_blk = WPAD // BLK

    def tg_kernel(i_ref, o_ref):
        i = pl.program_id(0)
        x = i_ref[...].T
        col = i * BLK + lax.broadcasted_iota(jnp.int32, (B, BLK), 1)
        q = col // STRIDE
        r = col - q * STRIDE
        invalid = (r >= FS) | (col >= W)
        o_ref[...] = jnp.where(invalid, TRASH, q * FS + x)

    return pl.pallas_call(
        tg_kernel,
        grid=(n_blk,),
        in_specs=[pl.BlockSpec((BLK, B), lambda i: (i, 0))],
        out_specs=pl.BlockSpec((B, BLK), lambda i: (0, i)),
        out_shape=jax.ShapeDtypeStruct((B, WPAD), jnp.int32),
    )(sp_t)


def _sc_histogram(targets_flat, B):
    b_per_w = B // NW
    mesh = plsc.VectorSubcoreMesh(core_axis_name="c", subcore_axis_name="s")

    @functools.partial(
        pl.kernel,
        mesh=mesh,
        out_type=jax.ShapeDtypeStruct((B * CW,), jnp.float32),
        compiler_params=pltpu.CompilerParams(
            needs_layout_passes=False, use_tc_tiling_on_sc=False),
        scratch_types=[
            pltpu.VMEM((WPAD,), jnp.int32),
            pltpu.VMEM((HP,), jnp.float32),
        ],
    )
    def hist_kernel(tg_hbm, out_hbm, idx_v, hist_v):
        wid = lax.axis_index("s") * 2 + lax.axis_index("c")
        ones = jnp.full((LANES,), 1.0, dtype=jnp.float32)
        zeros = jnp.zeros((LANES,), dtype=jnp.float32)

        def per_sample(s, _):
            row = wid * b_per_w + s
            pltpu.sync_copy(tg_hbm.at[pl.ds(row * WPAD, WPAD)], idx_v)

            def zero_chunk(j, _):
                hist_v[pl.ds(j * LANES, LANES)] = zeros
                return 0

            lax.fori_loop(0, N_HCHUNKS, zero_chunk, 0)

            def scatter_chunk(k, _):
                t = idx_v[pl.ds(k * LANES, LANES)]
                plsc.addupdate_scatter(hist_v, [t], ones)
                return 0

            lax.fori_loop(0, N_CHUNKS, scatter_chunk, 0)
            pltpu.sync_copy(hist_v.at[pl.ds(0, CW)],
                            out_hbm.at[pl.ds(row * CW, CW)])
            return 0

        lax.fori_loop(0, b_per_w, per_sample, 0)

    return hist_kernel(targets_flat)


def _tc_fm(counts, dense_t, emb_t, le_t, lin_W, bias):
    B = counts.shape[0]
    ND = dense_t.shape[0]
    BLK = 256
    n_blk = B // BLK
    hi = jax.lax.Precision.HIGHEST
    contract_last = (((1,), (1,)), ((), ()))

    def fm_kernel(c_ref, dn_ref, et_ref, le_ref, w_ref, b_ref, o_ref):
        et = et_ref[...]
        lw = le_ref[...] * w_ref[0:FIELDS, :]
        s = jnp.zeros((BLK, EMB_D), jnp.float32)
        q = jnp.zeros((BLK, EMB_D), jnp.float32)
        ln = jnp.zeros((BLK, 1), jnp.float32)
        for i in range(FIELDS):
            ci = c_ref[:, i * FS:(i + 1) * FS]
            ei = et[i]
            s = s + lax.dot_general(ci, ei, contract_last, precision=hi)
            q = q + lax.dot_general(ci, ei * ei, contract_last, precision=hi)
            ln = ln + lax.dot_general(ci, lw[i:i + 1, :], contract_last,
                                      precision=hi)
        cross = 0.5 * (jnp.sum(s * s, axis=1, keepdims=True)
                       - jnp.sum(q, axis=1, keepdims=True))
        dense_part = lax.dot_general(dn_ref[...], w_ref[FIELDS:, :],
                                     (((0,), (0,)), ((), ())), precision=hi)
        lin = ln + dense_part + b_ref[0, 0]
        o_ref[...] = jax.nn.sigmoid(lin + cross)

    return pl.pallas_call(
        fm_kernel,
        grid=(n_blk,),
        in_specs=[
            pl.BlockSpec((BLK, CW), lambda i: (i, 0)),
            pl.BlockSpec((ND, BLK), lambda i: (0, i)),
            pl.BlockSpec((FIELDS, EMB_D, FS), lambda i: (0, 0, 0)),
            pl.BlockSpec((FIELDS, FS), lambda i: (0, 0)),
            pl.BlockSpec((FIELDS + ND, 1), lambda i: (0, 0)),
            pl.BlockSpec((1, 1), lambda i: (0, 0)),
        ],
        out_specs=pl.BlockSpec((BLK, 1), lambda i: (i, 0)),
        out_shape=jax.ShapeDtypeStruct((B, 1), jnp.float32),
    )(counts, dense_t, emb_t, le_t, lin_W, bias)


def kernel(sparse_feat, dense_feat, linear_emb, emb, lin_W, lin_b):
    B = sparse_feat.shape[0]
    sp_t = sparse_feat.astype(jnp.int32).T
    dense_t = dense_feat.T
    emb_t = jnp.transpose(emb, (0, 2, 1))
    le_t = jnp.transpose(linear_emb, (0, 2, 1)).reshape(FIELDS, FS)
    bias = lin_b.reshape(1, 1)

    targets = _tc_targets(sp_t).reshape(B * WPAD)
    counts = _sc_histogram(targets, B).reshape(B, CW)
    return _tc_fm(counts, dense_t, emb_t, le_t, lin_W, bias)


# speedup vs baseline: 222.5924x; 1.2646x over previous
import functools

import jax
import jax.numpy as jnp
from jax import lax
from jax.experimental import pallas as pl
from jax.experimental.pallas import tpu as pltpu
from jax.experimental.pallas import tpu_sc as plsc

FIELDS = 26
FS = 100
EMB_D = 32
STRIDE = FS + 1
WIDTH = FIELDS * STRIDE
CW = FIELDS * FS
WPAD = 2688
HP = 2816
TRASH = CW
LANES = 16
N_CHUNKS = WPAD // LANES
N_HCHUNKS = HP // LANES
NW = 32


def _tc_targets(sp_t):
    W, B = sp_t.shape
    BLK = 128
    n_blk = WPAD // BLK

    def tg_kernel(i_ref, o_ref):
        i = pl.program_id(0)
        x = i_ref[...].T
        col = i * BLK + lax.broadcasted_iota(jnp.int32, (B, BLK), 1)
        q = col // STRIDE
        r = col - q * STRIDE
        invalid = (r >= FS) | (col >= W)
        o_ref[...] = jnp.where(invalid, TRASH, q * FS + x)

    return pl.pallas_call(
        tg_kernel,
        grid=(n_blk,),
        in_specs=[pl.BlockSpec((BLK, B), lambda i: (i, 0))],
        out_specs=pl.BlockSpec((B, BLK), lambda i: (0, i)),
        out_shape=jax.ShapeDtypeStruct((B, WPAD), jnp.int32),
    )(sp_t)


G = 8
GW = G * WPAD


def _sc_histogram(targets_flat, B):
    b_per_w = B // NW
    n_groups = b_per_w // G
    mesh = plsc.VectorSubcoreMesh(core_axis_name="c", subcore_axis_name="s")

    @functools.partial(
        pl.kernel,
        mesh=mesh,
        out_type=jax.ShapeDtypeStruct((B * WPAD,), jnp.float32),
        compiler_params=pltpu.CompilerParams(
            needs_layout_passes=False, use_tc_tiling_on_sc=False),
        scratch_types=[
            pltpu.VMEM((GW,), jnp.int32),
            pltpu.VMEM((GW,), jnp.float32),
        ],
    )
    def hist_kernel(tg_hbm, out_hbm, idx_v, hist_v):
        wid = lax.axis_index("s") * 2 + lax.axis_index("c")
        ones = jnp.full((LANES,), 1.0, dtype=jnp.float32)
        zeros = jnp.zeros((LANES,), dtype=jnp.float32)

        def per_group(g, _):
            base = (wid * b_per_w + g * G) * WPAD
            pltpu.sync_copy(tg_hbm.at[pl.ds(base, GW)], idx_v)

            def zero_chunk(j, _):
                hist_v[pl.ds(j * LANES, LANES)] = zeros
                return 0

            lax.fori_loop(0, GW // LANES, zero_chunk, 0, unroll=8)

            for j in range(G):
                def scatter_chunk(k, _, j=j):
                    t = idx_v[pl.ds(j * WPAD + k * LANES, LANES)]
                    plsc.addupdate_scatter(hist_v, [t + (j * WPAD)], ones)
                    return 0

                lax.fori_loop(0, N_CHUNKS, scatter_chunk, 0, unroll=8)

            pltpu.sync_copy(hist_v, out_hbm.at[pl.ds(base, GW)])
            return 0

        lax.fori_loop(0, n_groups, per_group, 0)

    return hist_kernel(targets_flat)


def _tc_fm(counts, dense_t, emb_t, le_t, lin_W, bias):
    B = counts.shape[0]
    ND = dense_t.shape[0]
    BLK = 256
    n_blk = B // BLK
    hi = jax.lax.Precision.HIGHEST
    contract_last = (((1,), (1,)), ((), ()))

    def fm_kernel(c_ref, dn_ref, et_ref, le_ref, w_ref, b_ref, o_ref):
        et = et_ref[...]
        lw = le_ref[...] * w_ref[0:FIELDS, :]
        s = jnp.zeros((BLK, EMB_D), jnp.float32)
        q = jnp.zeros((BLK, EMB_D), jnp.float32)
        ln = jnp.zeros((BLK, 1), jnp.float32)
        for i in range(FIELDS):
            ci = c_ref[:, i * FS:(i + 1) * FS]
            ei = et[i]
            s = s + lax.dot_general(ci, ei, contract_last, precision=hi)
            q = q + lax.dot_general(ci, ei * ei, contract_last, precision=hi)
            ln = ln + lax.dot_general(ci, lw[i:i + 1, :], contract_last,
                                      precision=hi)
        cross = 0.5 * (jnp.sum(s * s, axis=1, keepdims=True)
                       - jnp.sum(q, axis=1, keepdims=True))
        dense_part = lax.dot_general(dn_ref[...], w_ref[FIELDS:, :],
                                     (((0,), (0,)), ((), ())), precision=hi)
        lin = ln + dense_part + b_ref[0, 0]
        o_ref[...] = jax.nn.sigmoid(lin + cross)

    return pl.pallas_call(
        fm_kernel,
        grid=(n_blk,),
        in_specs=[
            pl.BlockSpec((BLK, WPAD), lambda i: (i, 0)),
            pl.BlockSpec((ND, BLK), lambda i: (0, i)),
            pl.BlockSpec((FIELDS, EMB_D, FS), lambda i: (0, 0, 0)),
            pl.BlockSpec((FIELDS, FS), lambda i: (0, 0)),
            pl.BlockSpec((FIELDS + ND, 1), lambda i: (0, 0)),
            pl.BlockSpec((1, 1), lambda i: (0, 0)),
        ],
        out_specs=pl.BlockSpec((BLK, 1), lambda i: (i, 0)),
        out_shape=jax.ShapeDtypeStruct((B, 1), jnp.float32),
    )(counts, dense_t, emb_t, le_t, lin_W, bias)


def kernel(sparse_feat, dense_feat, linear_emb, emb, lin_W, lin_b):
    B = sparse_feat.shape[0]
    sp_t = sparse_feat.astype(jnp.int32).T
    dense_t = dense_feat.T
    emb_t = jnp.transpose(emb, (0, 2, 1))
    le_t = jnp.transpose(linear_emb, (0, 2, 1)).reshape(FIELDS, FS)
    bias = lin_b.reshape(1, 1)

    targets = _tc_targets(sp_t).reshape(B * WPAD)
    counts = _sc_histogram(targets, B).reshape(B, WPAD)
    return _tc_fm(counts, dense_t, emb_t, le_t, lin_W, bias)


# speedup vs baseline: 234.0332x; 1.0514x over previous
import functools

import jax
import jax.numpy as jnp
from jax import lax
from jax.experimental import pallas as pl
from jax.experimental.pallas import tpu as pltpu
from jax.experimental.pallas import tpu_sc as plsc

FIELDS = 26
FS = 100
EMB_D = 32
STRIDE = FS + 1
WIDTH = FIELDS * STRIDE
CW = FIELDS * FS
WPAD = 2688
HP = 2816
TRASH = CW
LANES = 16
N_CHUNKS = WPAD // LANES
N_HCHUNKS = HP // LANES
NW = 32


def _tc_targets(sp_t):
    W, B = sp_t.shape
    BLK = 128
    n_blk = WPAD // BLK

    def tg_kernel(i_ref, o_ref):
        i = pl.program_id(0)
        x = i_ref[...].T
        col = i * BLK + lax.broadcasted_iota(jnp.int32, (B, BLK), 1)
        q = col // STRIDE
        r = col - q * STRIDE
        invalid = (r >= FS) | (col >= W)
        o_ref[...] = jnp.where(invalid, TRASH, q * FS + x)

    return pl.pallas_call(
        tg_kernel,
        grid=(n_blk,),
        in_specs=[pl.BlockSpec((BLK, B), lambda i: (i, 0))],
        out_specs=pl.BlockSpec((B, BLK), lambda i: (0, i)),
        out_shape=jax.ShapeDtypeStruct((B, WPAD), jnp.int32),
    )(sp_t)


G = 8
GW = G * WPAD


def _sc_histogram(targets_flat, B):
    b_per_w = B // NW
    n_groups = b_per_w // G
    mesh = plsc.VectorSubcoreMesh(core_axis_name="c", subcore_axis_name="s")

    @functools.partial(
        pl.kernel,
        mesh=mesh,
        out_type=jax.ShapeDtypeStruct((B * WPAD,), jnp.float32),
        compiler_params=pltpu.CompilerParams(
            needs_layout_passes=False, use_tc_tiling_on_sc=False),
        scratch_types=[
            pltpu.VMEM((2, GW), jnp.int32),
            pltpu.VMEM((2, GW), jnp.float32),
            pltpu.SemaphoreType.DMA,
            pltpu.SemaphoreType.DMA,
            pltpu.SemaphoreType.DMA,
            pltpu.SemaphoreType.DMA,
        ],
    )
    def hist_kernel(tg_hbm, out_hbm, idx_v, hist_v, si0, si1, so0, so1):
        wid = lax.axis_index("s") * 2 + lax.axis_index("c")
        ones = jnp.full((LANES,), 1.0, dtype=jnp.float32)
        zeros = jnp.zeros((LANES,), dtype=jnp.float32)
        sin = (si0, si1)
        sout = (so0, so1)
        base0 = wid * b_per_w * WPAD

        in_cp = [None] * n_groups
        out_cp = [None] * n_groups
        in_cp[0] = pltpu.async_copy(
            tg_hbm.at[pl.ds(base0, GW)], idx_v.at[0], sin[0])
        for g in range(n_groups):
            b = g % 2
            if g + 1 < n_groups:
                in_cp[g + 1] = pltpu.async_copy(
                    tg_hbm.at[pl.ds(base0 + (g + 1) * GW, GW)],
                    idx_v.at[(g + 1) % 2], sin[(g + 1) % 2])
            if g >= 2:
                out_cp[g - 2].wait()

            def zero_chunk(j, _, b=b):
                hist_v[b, pl.ds(j * LANES, LANES)] = zeros
                return 0

            lax.fori_loop(0, GW // LANES, zero_chunk, 0, unroll=8)
            in_cp[g].wait()

            for j in range(G):
                def scatter_chunk(k, _, b=b, j=j):
                    t = idx_v[b, pl.ds(j * WPAD + k * LANES, LANES)]
                    plsc.addupdate_scatter(
                        hist_v.at[b], [t + (j * WPAD)], ones)
                    return 0

                lax.fori_loop(0, N_CHUNKS, scatter_chunk, 0, unroll=8)

            out_cp[g] = pltpu.async_copy(
                hist_v.at[b], out_hbm.at[pl.ds(base0 + g * GW, GW)], sout[b])
        for g in range(max(0, n_groups - 2), n_groups):
            out_cp[g].wait()

    return hist_kernel(targets_flat)


def _tc_fm(counts, dense_t, emb_t, le_t, lin_W, bias):
    B = counts.shape[0]
    ND = dense_t.shape[0]
    BLK = 256
    n_blk = B // BLK
    hi = jax.lax.Precision.HIGHEST
    contract_last = (((1,), (1,)), ((), ()))

    def fm_kernel(c_ref, dn_ref, et_ref, le_ref, w_ref, b_ref, o_ref):
        et = et_ref[...]
        lw = le_ref[...] * w_ref[0:FIELDS, :]
        s = jnp.zeros((BLK, EMB_D), jnp.float32)
        q = jnp.zeros((BLK, EMB_D), jnp.float32)
        ln = jnp.zeros((BLK, 1), jnp.float32)
        for i in range(FIELDS):
            ci = c_ref[:, i * FS:(i + 1) * FS]
            ei = et[i]
            s = s + lax.dot_general(ci, ei, contract_last, precision=hi)
            q = q + lax.dot_general(ci, ei * ei, contract_last, precision=hi)
            ln = ln + lax.dot_general(ci, lw[i:i + 1, :], contract_last,
                                      precision=hi)
        cross = 0.5 * (jnp.sum(s * s, axis=1, keepdims=True)
                       - jnp.sum(q, axis=1, keepdims=True))
        dense_part = lax.dot_general(dn_ref[...], w_ref[FIELDS:, :],
                                     (((0,), (0,)), ((), ())), precision=hi)
        lin = ln + dense_part + b_ref[0, 0]
        o_ref[...] = jax.nn.sigmoid(lin + cross)

    return pl.pallas_call(
        fm_kernel,
        grid=(n_blk,),
        in_specs=[
            pl.BlockSpec((BLK, WPAD), lambda i: (i, 0)),
            pl.BlockSpec((ND, BLK), lambda i: (0, i)),
            pl.BlockSpec((FIELDS, EMB_D, FS), lambda i: (0, 0, 0)),
            pl.BlockSpec((FIELDS, FS), lambda i: (0, 0)),
            pl.BlockSpec((FIELDS + ND, 1), lambda i: (0, 0)),
            pl.BlockSpec((1, 1), lambda i: (0, 0)),
        ],
        out_specs=pl.BlockSpec((BLK, 1), lambda i: (i, 0)),
        out_shape=jax.ShapeDtypeStruct((B, 1), jnp.float32),
    )(counts, dense_t, emb_t, le_t, lin_W, bias)


def kernel(sparse_feat, dense_feat, linear_emb, emb, lin_W, lin_b):
    B = sparse_feat.shape[0]
    sp_t = sparse_feat.astype(jnp.int32).T
    dense_t = dense_feat.T
    emb_t = jnp.transpose(emb, (0, 2, 1))
    le_t = jnp.transpose(linear_emb, (0, 2, 1)).reshape(FIELDS, FS)
    bias = lin_b.reshape(1, 1)

    targets = _tc_targets(sp_t).reshape(B * WPAD)
    counts = _sc_histogram(targets, B).reshape(B, WPAD)
    return _tc_fm(counts, dense_t, emb_t, le_t, lin_W, bias)
